# baseline (speedup 1.0000x reference)
import functools

import jax
import jax.numpy as jnp
from jax import lax
from jax.experimental import pallas as pl
from jax.experimental.pallas import tpu as pltpu
from jax.experimental.pallas import tpu_sc as plsc

N_NODES = 10000
N_EDGES = 320000
D = 128

NC = 2
NS = 16
NW = NC * NS
E_PER_W = N_EDGES // NW
CHUNK = 104
N_FULL = E_PER_W // CHUNK
TAIL = E_PER_W - N_FULL * CHUNK
ZCH = 128
N_PAD = 10240
ROWS_PER_TILE = N_PAD // NS
HIST_W = 16

_mesh = plsc.VectorSubcoreMesh(core_axis_name="c", subcore_axis_name="s")


def _zero_vmem_2d(buf, nrows, width):
    def row(i, _):
        for k in range(width // 16):
            buf[i, pl.ds(k * 16, 16)] = jnp.zeros((16,), jnp.float32)
        return 0

    lax.fori_loop(0, nrows, row, 0, unroll=False)


def _fill_ones_2d(buf, nrows, width):
    def row(i, _):
        for k in range(width // 16):
            buf[i, pl.ds(k * 16, 16)] = jnp.ones((16,), jnp.float32)
        return 0

    lax.fori_loop(0, nrows, row, 0, unroll=False)


_DEG_KERNEL_CFG = dict(
    out_type=jax.ShapeDtypeStruct((NC, N_PAD, HIST_W), jnp.float32),
    mesh=_mesh,
    scratch_types=[
        pltpu.VMEM((ZCH, HIST_W), jnp.float32),
        pltpu.VMEM((2, CHUNK), jnp.int32),
        pltpu.VMEM((TAIL,), jnp.int32),
        pltpu.VMEM_SHARED((N_PAD, HIST_W), jnp.float32),
        pltpu.SemaphoreType.DMA,
        pltpu.SemaphoreType.DMA,
    ],
)


def _deg_body(dst_hbm, hist_hbm, ones_v, idx_d, idx_dt, hist_sp, ssem, isem):
    c = lax.axis_index("c")
    s = lax.axis_index("s")
    wid = s * NC + c
    base = wid * E_PER_W

    _zero_vmem_2d(ones_v, ZCH, HIST_W)
    r0 = s * ROWS_PER_TILE
    for k in range(5):
        pltpu.sync_copy(
            ones_v,
            hist_sp.at[pl.ds(r0 + k * ZCH, ZCH)],
        )
    _fill_ones_2d(ones_v, ZCH, HIST_W)
    plsc.subcore_barrier()

    pltpu.sync_copy(dst_hbm.at[pl.ds(base, CHUNK)], idx_d.at[0])

    def pair(k, _):
        for b in (0, 1):
            j = 2 * k + b
            nb = 1 - b

            @pl.when(j >= 1)
            def _():
                pltpu.make_async_copy(
                    ones_v.at[pl.ds(0, CHUNK)],
                    hist_sp.at[idx_d.at[nb]], ssem).wait()

            @pl.when(j < N_FULL - 1)
            def _():
                off2 = base + (j + 1) * CHUNK
                pltpu.async_copy(
                    dst_hbm.at[pl.ds(off2, CHUNK)], idx_d.at[nb], isem)

            pltpu.async_copy(ones_v.at[pl.ds(0, CHUNK)], hist_sp.at[idx_d.at[b]],
                             ssem, add=True)

            @pl.when(j < N_FULL - 1)
            def _():
                off2 = base + (j + 1) * CHUNK
                pltpu.make_async_copy(
                    dst_hbm.at[pl.ds(off2, CHUNK)], idx_d.at[nb], isem).wait()
        return 0

    lax.fori_loop(0, N_FULL // 2, pair, 0, unroll=False)
    pltpu.make_async_copy(ones_v.at[pl.ds(0, CHUNK)],
                          hist_sp.at[idx_d.at[1]], ssem).wait()

    pltpu.sync_copy(dst_hbm.at[pl.ds(base + N_FULL * CHUNK, TAIL)], idx_dt)
    pltpu.sync_copy(ones_v.at[pl.ds(0, TAIL)], hist_sp.at[idx_dt], add=True)

    plsc.subcore_barrier()
    pltpu.sync_copy(
        hist_sp.at[pl.ds(r0, ROWS_PER_TILE)],
        hist_hbm.at[c, pl.ds(r0, ROWS_PER_TILE)],
    )


_MSG_KERNEL_CFG = dict(
    out_type=jax.ShapeDtypeStruct((NC, N_PAD, D), jnp.float32),
    mesh=_mesh,
    scratch_types=[
        pltpu.VMEM((3, 2, CHUNK), jnp.int32),
        pltpu.VMEM((TAIL,), jnp.int32),
        pltpu.VMEM((TAIL,), jnp.int32),
        pltpu.VMEM((3, CHUNK, D), jnp.float32),
        pltpu.VMEM((TAIL, D), jnp.float32),
        pltpu.VMEM_SHARED((N_PAD, D), jnp.float32),
        pltpu.SemaphoreType.DMA,
        pltpu.SemaphoreType.DMA,
        pltpu.SemaphoreType.DMA,
        pltpu.SemaphoreType.DMA,
        pltpu.SemaphoreType.DMA,
    ],
)


def _msg_body(g_hbm, src_hbm, dst_hbm, part_hbm,
              idx2, idx_st, idx_dt, rows, rows_t, acc,
              gsem0, gsem1, gsem2, ssem, isem):
    gsems = (gsem0, gsem1, gsem2)
    c = lax.axis_index("c")
    s = lax.axis_index("s")
    wid = s * NC + c
    base = wid * E_PER_W

    for b in (0, 2):
        _zero_vmem_2d(rows.at[b], CHUNK, D)
    _zero_vmem_2d(rows_t, TAIL, D)
    for o in (0, 16, 32, 48, 64, 80, CHUNK - 16):
        idx2[2, 1, pl.ds(o, 16)] = jnp.zeros((16,), jnp.int32)
    r0 = s * ROWS_PER_TILE
    for k in range(6):
        pltpu.sync_copy(rows.at[0], acc.at[pl.ds(r0 + k * CHUNK, CHUNK)])
    pltpu.sync_copy(rows_t, acc.at[pl.ds(r0 + 6 * CHUNK, TAIL)])
    plsc.subcore_barrier()

    pltpu.async_copy(rows.at[2], acc.at[idx2.at[2, 1]], ssem, add=True)
    pltpu.sync_copy(src_hbm.at[pl.ds(base, CHUNK)], idx2.at[0, 0])
    pltpu.sync_copy(dst_hbm.at[pl.ds(base, CHUNK)], idx2.at[0, 1])
    pltpu.async_copy(g_hbm.at[idx2.at[0, 0]], rows.at[0], gsems[0])
    pltpu.sync_copy(src_hbm.at[pl.ds(base + CHUNK, CHUNK)], idx2.at[1, 0])
    pltpu.sync_copy(dst_hbm.at[pl.ds(base + CHUNK, CHUNK)], idx2.at[1, 1])
    pltpu.async_copy(g_hbm.at[idx2.at[1, 0]], rows.at[1], gsems[1])

    def triple(k, _):
        for b in (0, 1, 2):
            j = 3 * k + b
            pb = (b + 2) % 3
            pltpu.make_async_copy(
                rows.at[pb], acc.at[idx2.at[pb, 1]], ssem).wait()

            @pl.when(j < N_FULL - 2)
            def _():
                off2 = base + (j + 2) * CHUNK
                pltpu.async_copy(
                    src_hbm.at[pl.ds(off2, CHUNK)], idx2.at[pb, 0], isem)
                pltpu.async_copy(
                    dst_hbm.at[pl.ds(off2, CHUNK)], idx2.at[pb, 1], isem)

            pltpu.make_async_copy(
                g_hbm.at[idx2.at[b, 0]], rows.at[b], gsems[b]).wait()
            pltpu.async_copy(rows.at[b], acc.at[idx2.at[b, 1]], ssem,
                             add=True)

            @pl.when(j < N_FULL - 2)
            def _():
                off2 = base + (j + 2) * CHUNK
                pltpu.make_async_copy(
                    src_hbm.at[pl.ds(off2, CHUNK)], idx2.at[pb, 0],
                    isem).wait()
                pltpu.make_async_copy(
                    dst_hbm.at[pl.ds(off2, CHUNK)], idx2.at[pb, 1],
                    isem).wait()
                pltpu.async_copy(g_hbm.at[idx2.at[pb, 0]], rows.at[pb],
                                 gsems[pb])
        return 0

    lax.fori_loop(0, N_FULL // 3, triple, 0, unroll=False)

    pltpu.make_async_copy(rows.at[2], acc.at[idx2.at[2, 1]], ssem).wait()

    off = base + N_FULL * CHUNK
    pltpu.sync_copy(src_hbm.at[pl.ds(off, TAIL)], idx_st)
    pltpu.sync_copy(dst_hbm.at[pl.ds(off, TAIL)], idx_dt)
    pltpu.async_copy(g_hbm.at[idx_st], rows_t, gsems[0]).wait()
    pltpu.sync_copy(rows_t, acc.at[idx_dt], add=True)

    plsc.subcore_barrier()
    pltpu.sync_copy(
        acc.at[pl.ds(r0, ROWS_PER_TILE)],
        part_hbm.at[c, pl.ds(r0, ROWS_PER_TILE)],
    )


_deg_kernel = pl.kernel(_deg_body, **_DEG_KERNEL_CFG)
_msg_kernel = pl.kernel(_msg_body, **_MSG_KERNEL_CFG)


ROW_BLK = 1000
GRID = N_NODES // ROW_BLK


def _dinv_from_hist(h0, h1):
    deg = 1.0 + h0[0, :, 0:1] + h1[0, :, 0:1]
    return lax.rsqrt(deg)


def _scale_matmul_body(x_ref, w_ref, h0_ref, h1_ref, o_ref):
    dinv = _dinv_from_hist(h0_ref[...], h1_ref[...])
    o_ref[...] = dinv * jnp.dot(
        x_ref[...], w_ref[...], preferred_element_type=jnp.float32
    )


def _scaled_matmul(x, w, hist):
    return pl.pallas_call(
        _scale_matmul_body,
        grid=(GRID,),
        in_specs=[
            pl.BlockSpec((ROW_BLK, D), lambda i: (i, 0)),
            pl.BlockSpec((D, D), lambda i: (0, 0)),
            pl.BlockSpec((1, ROW_BLK, HIST_W), lambda i: (0, i, 0)),
            pl.BlockSpec((1, ROW_BLK, HIST_W), lambda i: (1, i, 0)),
        ],
        out_specs=pl.BlockSpec((ROW_BLK, D), lambda i: (i, 0)),
        out_shape=jax.ShapeDtypeStruct((N_NODES, D), jnp.float32),
    )(x, w, hist, hist)


def _combine_relu_matmul_body(p_ref, g_ref, w_ref, b_ref, h0_ref, h1_ref,
                              o_ref):
    dinv = _dinv_from_hist(h0_ref[...], h1_ref[...])
    tot = p_ref[0] + p_ref[1] + g_ref[...]
    h = jnp.maximum(dinv * tot + b_ref[...], 0.0)
    o_ref[...] = dinv * jnp.dot(
        h, w_ref[...], preferred_element_type=jnp.float32
    )


def _combine_relu_matmul(part, g, w, b, hist):
    return pl.pallas_call(
        _combine_relu_matmul_body,
        grid=(GRID,),
        in_specs=[
            pl.BlockSpec((NC, ROW_BLK, D), lambda i: (0, i, 0)),
            pl.BlockSpec((ROW_BLK, D), lambda i: (i, 0)),
            pl.BlockSpec((D, D), lambda i: (0, 0)),
            pl.BlockSpec((1, D), lambda i: (0, 0)),
            pl.BlockSpec((1, ROW_BLK, HIST_W), lambda i: (0, i, 0)),
            pl.BlockSpec((1, ROW_BLK, HIST_W), lambda i: (1, i, 0)),
        ],
        out_specs=pl.BlockSpec((ROW_BLK, D), lambda i: (i, 0)),
        out_shape=jax.ShapeDtypeStruct((N_NODES, D), jnp.float32),
    )(part, g, w, b.reshape(1, D), hist, hist)


def _final_combine_body(p_ref, g_ref, b_ref, h0_ref, h1_ref, o_ref):
    dinv = _dinv_from_hist(h0_ref[...], h1_ref[...])
    tot = p_ref[0] + p_ref[1] + g_ref[...]
    o_ref[...] = dinv * tot + b_ref[...]


def _final_combine(part, g, b, hist):
    return pl.pallas_call(
        _final_combine_body,
        grid=(GRID,),
        in_specs=[
            pl.BlockSpec((NC, ROW_BLK, D), lambda i: (0, i, 0)),
            pl.BlockSpec((ROW_BLK, D), lambda i: (i, 0)),
            pl.BlockSpec((1, D), lambda i: (0, 0)),
            pl.BlockSpec((1, ROW_BLK, HIST_W), lambda i: (0, i, 0)),
            pl.BlockSpec((1, ROW_BLK, HIST_W), lambda i: (1, i, 0)),
        ],
        out_specs=pl.BlockSpec((ROW_BLK, D), lambda i: (i, 0)),
        out_shape=jax.ShapeDtypeStruct((N_NODES, D), jnp.float32),
    )(part, g, b.reshape(1, D), hist, hist)


def _fake_hist(dst):
    cnt = jax.ops.segment_sum(jnp.ones_like(dst, jnp.float32), dst,
                              num_segments=N_PAD)
    h0 = jnp.broadcast_to(cnt[:, None], (N_PAD, HIST_W))
    return jnp.stack([h0, jnp.zeros_like(h0)])


def _fake_msg(g, src, dst):
    p = jax.ops.segment_sum(g[src], dst, num_segments=N_PAD)
    return jnp.stack([p, jnp.zeros_like(p)])


def kernel(x, edge_index, W1, b1, W2, b2):
    ei = edge_index.astype(jnp.int32)
    src = ei[0]
    dst = ei[1]

    hist = _deg_kernel(dst)
    g1 = _scaled_matmul(x, W1, hist)
    p1 = _msg_kernel(g1, src, dst)
    g2 = _combine_relu_matmul(p1, g1, W2, b1, hist)
    p2 = _msg_kernel(g2, src, dst)
    return _final_combine(p2, g2, b2, hist)


# speedup vs baseline: 1.1138x; 1.1138x over previous
import functools

import jax
import jax.numpy as jnp
from jax import lax
from jax.experimental import pallas as pl
from jax.experimental.pallas import tpu as pltpu
from jax.experimental.pallas import tpu_sc as plsc

N_NODES = 10000
N_EDGES = 320000
D = 128

NC = 2
NS = 16
NW = NC * NS
E_PER_W = N_EDGES // NW
CHUNK = 104
N_FULL = E_PER_W // CHUNK
TAIL = E_PER_W - N_FULL * CHUNK
ZCH = 128
N_PAD = 10240
ROWS_PER_TILE = N_PAD // NS
HIST_W = 16

_mesh = plsc.VectorSubcoreMesh(core_axis_name="c", subcore_axis_name="s")


def _zero_vmem_2d(buf, nrows, width):
    def row(i, _):
        for k in range(width // 16):
            buf[i, pl.ds(k * 16, 16)] = jnp.zeros((16,), jnp.float32)
        return 0

    lax.fori_loop(0, nrows, row, 0, unroll=False)


def _fill_ones_2d(buf, nrows, width):
    def row(i, _):
        for k in range(width // 16):
            buf[i, pl.ds(k * 16, 16)] = jnp.ones((16,), jnp.float32)
        return 0

    lax.fori_loop(0, nrows, row, 0, unroll=False)


GSZ = 8
N_GRP = N_FULL // GSZ


_DEG_KERNEL_CFG = dict(
    out_type=jax.ShapeDtypeStruct((NC, N_PAD, HIST_W), jnp.float32),
    mesh=_mesh,
    scratch_types=[
        pltpu.VMEM((ZCH, HIST_W), jnp.float32),
        pltpu.VMEM((3, GSZ, CHUNK), jnp.int32),
        pltpu.VMEM((TAIL,), jnp.int32),
        pltpu.VMEM_SHARED((N_PAD, HIST_W), jnp.float32),
        pltpu.SemaphoreType.DMA,
        pltpu.SemaphoreType.DMA,
        pltpu.SemaphoreType.DMA,
    ],
)


def _deg_body(dst_hbm, hist_hbm, ones_v, idx_d, idx_dt, hist_sp,
              ssem0, ssem1, isem):
    ssems = (ssem0, ssem1)
    c = lax.axis_index("c")
    s = lax.axis_index("s")
    wid = s * NC + c
    base = wid * E_PER_W

    _zero_vmem_2d(ones_v, ZCH, HIST_W)
    r0 = s * ROWS_PER_TILE
    for k in range(5):
        pltpu.sync_copy(
            ones_v,
            hist_sp.at[pl.ds(r0 + k * ZCH, ZCH)],
        )
    _fill_ones_2d(ones_v, ZCH, HIST_W)
    plsc.subcore_barrier()

    src_rows = ones_v.at[pl.ds(0, CHUNK)]

    def load_group(g, bg):
        for m in range(GSZ):
            off = base + (g * GSZ + m) * CHUNK
            pltpu.async_copy(dst_hbm.at[pl.ds(off, CHUNK)],
                             idx_d.at[bg, m], isem)

    def wait_group(g, bg):
        for m in range(GSZ):
            off = base + (g * GSZ + m) * CHUNK
            pltpu.make_async_copy(dst_hbm.at[pl.ds(off, CHUNK)],
                                  idx_d.at[bg, m], isem).wait()

    load_group(0, 0)
    wait_group(0, 0)

    def six(k, _):
        for i in range(6):
            g = 6 * k + i
            bg = i % 3
            sp = i % 2

            @pl.when(g >= 2)
            def _():
                for m in range(GSZ):
                    pltpu.make_async_copy(
                        src_rows, hist_sp.at[idx_d.at[bg, m]],
                        ssems[sp]).wait()

            @pl.when(g >= 1)
            def _():
                wait_group(g, bg)

            for m in range(GSZ):
                pltpu.async_copy(src_rows, hist_sp.at[idx_d.at[bg, m]],
                                 ssems[sp], add=True)

            @pl.when(g < N_GRP - 1)
            def _():
                load_group(g + 1, (i + 1) % 3)
        return 0

    lax.fori_loop(0, N_GRP // 6, six, 0, unroll=False)

    for sp, bg in ((0, 1), (1, 2)):
        for m in range(GSZ):
            pltpu.make_async_copy(
                src_rows, hist_sp.at[idx_d.at[bg, m]], ssems[sp]).wait()

    pltpu.sync_copy(dst_hbm.at[pl.ds(base + N_FULL * CHUNK, TAIL)], idx_dt)
    pltpu.sync_copy(ones_v.at[pl.ds(0, TAIL)], hist_sp.at[idx_dt], add=True)

    plsc.subcore_barrier()
    pltpu.sync_copy(
        hist_sp.at[pl.ds(r0, ROWS_PER_TILE)],
        hist_hbm.at[c, pl.ds(r0, ROWS_PER_TILE)],
    )


_MSG_KERNEL_CFG = dict(
    out_type=jax.ShapeDtypeStruct((NC, N_PAD, D), jnp.float32),
    mesh=_mesh,
    scratch_types=[
        pltpu.VMEM((3, 2, CHUNK), jnp.int32),
        pltpu.VMEM((TAIL,), jnp.int32),
        pltpu.VMEM((TAIL,), jnp.int32),
        pltpu.VMEM((3, CHUNK, D), jnp.float32),
        pltpu.VMEM((TAIL, D), jnp.float32),
        pltpu.VMEM_SHARED((N_PAD, D), jnp.float32),
        pltpu.SemaphoreType.DMA,
        pltpu.SemaphoreType.DMA,
        pltpu.SemaphoreType.DMA,
        pltpu.SemaphoreType.DMA,
        pltpu.SemaphoreType.DMA,
    ],
)


def _msg_body(g_hbm, src_hbm, dst_hbm, part_hbm,
              idx2, idx_st, idx_dt, rows, rows_t, acc,
              gsem0, gsem1, gsem2, ssem, isem):
    gsems = (gsem0, gsem1, gsem2)
    c = lax.axis_index("c")
    s = lax.axis_index("s")
    wid = s * NC + c
    base = wid * E_PER_W

    for b in (0, 2):
        _zero_vmem_2d(rows.at[b], CHUNK, D)
    _zero_vmem_2d(rows_t, TAIL, D)
    for o in (0, 16, 32, 48, 64, 80, CHUNK - 16):
        idx2[2, 1, pl.ds(o, 16)] = jnp.zeros((16,), jnp.int32)
    r0 = s * ROWS_PER_TILE
    for k in range(6):
        pltpu.sync_copy(rows.at[0], acc.at[pl.ds(r0 + k * CHUNK, CHUNK)])
    pltpu.sync_copy(rows_t, acc.at[pl.ds(r0 + 6 * CHUNK, TAIL)])
    plsc.subcore_barrier()

    pltpu.async_copy(rows.at[2], acc.at[idx2.at[2, 1]], ssem, add=True)
    pltpu.sync_copy(src_hbm.at[pl.ds(base, CHUNK)], idx2.at[0, 0])
    pltpu.sync_copy(dst_hbm.at[pl.ds(base, CHUNK)], idx2.at[0, 1])
    pltpu.async_copy(g_hbm.at[idx2.at[0, 0]], rows.at[0], gsems[0])
    pltpu.sync_copy(src_hbm.at[pl.ds(base + CHUNK, CHUNK)], idx2.at[1, 0])
    pltpu.sync_copy(dst_hbm.at[pl.ds(base + CHUNK, CHUNK)], idx2.at[1, 1])
    pltpu.async_copy(g_hbm.at[idx2.at[1, 0]], rows.at[1], gsems[1])

    def triple(k, _):
        for b in (0, 1, 2):
            j = 3 * k + b
            pb = (b + 2) % 3
            pltpu.make_async_copy(
                rows.at[pb], acc.at[idx2.at[pb, 1]], ssem).wait()

            @pl.when(j < N_FULL - 2)
            def _():
                off2 = base + (j + 2) * CHUNK
                pltpu.async_copy(
                    src_hbm.at[pl.ds(off2, CHUNK)], idx2.at[pb, 0], isem)
                pltpu.async_copy(
                    dst_hbm.at[pl.ds(off2, CHUNK)], idx2.at[pb, 1], isem)

            pltpu.make_async_copy(
                g_hbm.at[idx2.at[b, 0]], rows.at[b], gsems[b]).wait()
            pltpu.async_copy(rows.at[b], acc.at[idx2.at[b, 1]], ssem,
                             add=True)

            @pl.when(j < N_FULL - 2)
            def _():
                off2 = base + (j + 2) * CHUNK
                pltpu.make_async_copy(
                    src_hbm.at[pl.ds(off2, CHUNK)], idx2.at[pb, 0],
                    isem).wait()
                pltpu.make_async_copy(
                    dst_hbm.at[pl.ds(off2, CHUNK)], idx2.at[pb, 1],
                    isem).wait()
                pltpu.async_copy(g_hbm.at[idx2.at[pb, 0]], rows.at[pb],
                                 gsems[pb])
        return 0

    lax.fori_loop(0, N_FULL // 3, triple, 0, unroll=False)

    pltpu.make_async_copy(rows.at[2], acc.at[idx2.at[2, 1]], ssem).wait()

    off = base + N_FULL * CHUNK
    pltpu.sync_copy(src_hbm.at[pl.ds(off, TAIL)], idx_st)
    pltpu.sync_copy(dst_hbm.at[pl.ds(off, TAIL)], idx_dt)
    pltpu.async_copy(g_hbm.at[idx_st], rows_t, gsems[0]).wait()
    pltpu.sync_copy(rows_t, acc.at[idx_dt], add=True)

    plsc.subcore_barrier()
    pltpu.sync_copy(
        acc.at[pl.ds(r0, ROWS_PER_TILE)],
        part_hbm.at[c, pl.ds(r0, ROWS_PER_TILE)],
    )


_deg_kernel = pl.kernel(_deg_body, **_DEG_KERNEL_CFG)
_msg_kernel = pl.kernel(_msg_body, **_MSG_KERNEL_CFG)


ROW_BLK = 1000
GRID = N_NODES // ROW_BLK


def _dinv_from_hist(h0, h1):
    deg = 1.0 + h0[0, :, 0:1] + h1[0, :, 0:1]
    return lax.rsqrt(deg)


def _scale_matmul_body(x_ref, w_ref, h0_ref, h1_ref, o_ref):
    dinv = _dinv_from_hist(h0_ref[...], h1_ref[...])
    o_ref[...] = dinv * jnp.dot(
        x_ref[...], w_ref[...], preferred_element_type=jnp.float32
    )


def _scaled_matmul(x, w, hist):
    return pl.pallas_call(
        _scale_matmul_body,
        grid=(GRID,),
        in_specs=[
            pl.BlockSpec((ROW_BLK, D), lambda i: (i, 0)),
            pl.BlockSpec((D, D), lambda i: (0, 0)),
            pl.BlockSpec((1, ROW_BLK, HIST_W), lambda i: (0, i, 0)),
            pl.BlockSpec((1, ROW_BLK, HIST_W), lambda i: (1, i, 0)),
        ],
        out_specs=pl.BlockSpec((ROW_BLK, D), lambda i: (i, 0)),
        out_shape=jax.ShapeDtypeStruct((N_NODES, D), jnp.float32),
    )(x, w, hist, hist)


def _combine_relu_matmul_body(p_ref, g_ref, w_ref, b_ref, h0_ref, h1_ref,
                              o_ref):
    dinv = _dinv_from_hist(h0_ref[...], h1_ref[...])
    tot = p_ref[0] + p_ref[1] + g_ref[...]
    h = jnp.maximum(dinv * tot + b_ref[...], 0.0)
    o_ref[...] = dinv * jnp.dot(
        h, w_ref[...], preferred_element_type=jnp.float32
    )


def _combine_relu_matmul(part, g, w, b, hist):
    return pl.pallas_call(
        _combine_relu_matmul_body,
        grid=(GRID,),
        in_specs=[
            pl.BlockSpec((NC, ROW_BLK, D), lambda i: (0, i, 0)),
            pl.BlockSpec((ROW_BLK, D), lambda i: (i, 0)),
            pl.BlockSpec((D, D), lambda i: (0, 0)),
            pl.BlockSpec((1, D), lambda i: (0, 0)),
            pl.BlockSpec((1, ROW_BLK, HIST_W), lambda i: (0, i, 0)),
            pl.BlockSpec((1, ROW_BLK, HIST_W), lambda i: (1, i, 0)),
        ],
        out_specs=pl.BlockSpec((ROW_BLK, D), lambda i: (i, 0)),
        out_shape=jax.ShapeDtypeStruct((N_NODES, D), jnp.float32),
    )(part, g, w, b.reshape(1, D), hist, hist)


def _final_combine_body(p_ref, g_ref, b_ref, h0_ref, h1_ref, o_ref):
    dinv = _dinv_from_hist(h0_ref[...], h1_ref[...])
    tot = p_ref[0] + p_ref[1] + g_ref[...]
    o_ref[...] = dinv * tot + b_ref[...]


def _final_combine(part, g, b, hist):
    return pl.pallas_call(
        _final_combine_body,
        grid=(GRID,),
        in_specs=[
            pl.BlockSpec((NC, ROW_BLK, D), lambda i: (0, i, 0)),
            pl.BlockSpec((ROW_BLK, D), lambda i: (i, 0)),
            pl.BlockSpec((1, D), lambda i: (0, 0)),
            pl.BlockSpec((1, ROW_BLK, HIST_W), lambda i: (0, i, 0)),
            pl.BlockSpec((1, ROW_BLK, HIST_W), lambda i: (1, i, 0)),
        ],
        out_specs=pl.BlockSpec((ROW_BLK, D), lambda i: (i, 0)),
        out_shape=jax.ShapeDtypeStruct((N_NODES, D), jnp.float32),
    )(part, g, b.reshape(1, D), hist, hist)


def _fake_hist(dst):
    cnt = jax.ops.segment_sum(jnp.ones_like(dst, jnp.float32), dst,
                              num_segments=N_PAD)
    h0 = jnp.broadcast_to(cnt[:, None], (N_PAD, HIST_W))
    return jnp.stack([h0, jnp.zeros_like(h0)])


def _fake_msg(g, src, dst):
    p = jax.ops.segment_sum(g[src], dst, num_segments=N_PAD)
    return jnp.stack([p, jnp.zeros_like(p)])


def kernel(x, edge_index, W1, b1, W2, b2):
    ei = edge_index.astype(jnp.int32)
    src = ei[0]
    dst = ei[1]

    hist = _deg_kernel(dst)
    g1 = _scaled_matmul(x, W1, hist)
    p1 = _msg_kernel(g1, src, dst)
    g2 = _combine_relu_matmul(p1, g1, W2, b1, hist)
    p2 = _msg_kernel(g2, src, dst)
    return _final_combine(p2, g2, b2, hist)
